# baseline (speedup 1.0000x reference)
import functools

import jax
import jax.numpy as jnp
from jax import lax
from jax.experimental import pallas as pl
from jax.experimental.pallas import tpu as pltpu
from jax.experimental.pallas import tpu_sc as plsc

N = 10000
E = 320000
D = 128
NC = 2
NS = 16
NW = NC * NS
K = 64
C = 160
EP = NW * C * K
NB = 4
SB = 40
NSB = C // SB
WD = 16
NP = 10240
RPP = NP // NS

_mesh = plsc.VectorSubcoreMesh(core_axis_name="c", subcore_axis_name="s")


KD = 80
CD = E // (NW * KD)


@functools.partial(
    pl.kernel,
    out_type=jax.ShapeDtypeStruct((NC, NP, D), jnp.float32),
    mesh=_mesh,
    scratch_types=[
        pltpu.VMEM_SHARED((NP, D), jnp.float32),
        pltpu.VMEM((CD, KD), jnp.int32),
        pltpu.VMEM((KD, D), jnp.float32),
    ],
)
def _deg_sc(dst_hbm, zeros_hbm, ones_hbm, out_hbm, acc, didx, ones_v):
    c = lax.axis_index("c")
    s = lax.axis_index("s")
    wid = s * NC + c
    pltpu.sync_copy(zeros_hbm.at[pl.ds(s * RPP, RPP)], acc.at[pl.ds(s * RPP, RPP)])
    pltpu.sync_copy(ones_hbm, ones_v)
    pltpu.sync_copy(dst_hbm.at[wid], didx)
    plsc.subcore_barrier()

    def body(j, carry):
        pltpu.sync_copy(ones_v, acc.at[didx.at[j]], add=True)
        return carry

    lax.fori_loop(0, CD, body, 0)
    plsc.subcore_barrier()
    pltpu.sync_copy(acc.at[pl.ds(s * RPP, RPP)], out_hbm.at[c, pl.ds(s * RPP, RPP)])


@functools.partial(
    pl.kernel,
    out_type=jax.ShapeDtypeStruct((NC, NP, D), jnp.float32),
    mesh=_mesh,
    scratch_types=[
        pltpu.VMEM_SHARED((NP, D), jnp.float32),
        pltpu.VMEM((SB, K), jnp.int32),
        pltpu.VMEM((SB, K), jnp.int32),
        pltpu.VMEM((NB, K, D), jnp.float32),
        [pltpu.SemaphoreType.DMA] * NB,
        [pltpu.SemaphoreType.DMA] * NB,
    ],
)
def _agg_sc(y_hbm, src_hbm, dst_hbm, zeros_hbm, out_hbm, acc, sidx, didx, rows,
            gsems, ssems):
    c = lax.axis_index("c")
    s = lax.axis_index("s")
    wid = s * NC + c
    pltpu.sync_copy(zeros_hbm.at[pl.ds(s * RPP, RPP)], acc.at[pl.ds(s * RPP, RPP)])
    plsc.subcore_barrier()

    def gwait(b):
        pltpu.make_async_copy(y_hbm.at[sidx.at[0]], rows.at[b], gsems[b]).wait()

    def swait(b):
        pltpu.make_async_copy(y_hbm.at[sidx.at[0]], rows.at[b], ssems[b]).wait()

    for t in range(NSB):
        pltpu.sync_copy(src_hbm.at[wid * NSB + t], sidx)
        pltpu.sync_copy(dst_hbm.at[wid * NSB + t], didx)
        for b in range(NB):
            pltpu.async_copy(y_hbm.at[sidx.at[b]], rows.at[b], gsems[b])

        def body(g, carry):
            for b in range(NB):
                j = g * NB + b
                gwait(b)
                pltpu.async_copy(rows.at[b], acc.at[didx.at[j]], ssems[b], add=True)
            for b in range(NB):
                j = (g + 1) * NB + b
                swait(b)
                pltpu.async_copy(y_hbm.at[sidx.at[j]], rows.at[b], gsems[b])
            return carry

        lax.fori_loop(0, SB // NB - 1, body, 0)
        for b in range(NB):
            j = SB - NB + b
            gwait(b)
            pltpu.async_copy(rows.at[b], acc.at[didx.at[j]], ssems[b], add=True)
        for b in range(NB):
            swait(b)

    plsc.subcore_barrier()
    pltpu.sync_copy(acc.at[pl.ds(s * RPP, RPP)], out_hbm.at[c, pl.ds(s * RPP, RPP)])


BN = 1000


def _tc1_body(x_ref, w_ref, degp_ref, dinv_ref, y_ref):
    deg = 1.0 + degp_ref[0, :, :1] + degp_ref[1, :, :1]
    dinv = lax.rsqrt(deg)
    dinv_ref[...] = jnp.broadcast_to(dinv, (BN, WD))
    xw = jnp.dot(x_ref[...], w_ref[...], precision=lax.Precision.HIGHEST,
                 preferred_element_type=jnp.float32)
    y_ref[...] = xw * dinv


_tc1 = pl.pallas_call(
    _tc1_body,
    grid=(N // BN,),
    in_specs=[
        pl.BlockSpec((BN, D), lambda i: (i, 0)),
        pl.BlockSpec((D, D), lambda i: (0, 0)),
        pl.BlockSpec((NC, BN, D), lambda i: (0, i, 0)),
    ],
    out_specs=[
        pl.BlockSpec((BN, WD), lambda i: (i, 0)),
        pl.BlockSpec((BN, D), lambda i: (i, 0)),
    ],
    out_shape=[
        jax.ShapeDtypeStruct((N, WD), jnp.float32),
        jax.ShapeDtypeStruct((N, D), jnp.float32),
    ],
)


def _tc2_body(aggp_ref, y1_ref, dinv_ref, b1_ref, w2_ref, y2_ref):
    dinv = dinv_ref[:, :1]
    pre = (aggp_ref[0] + aggp_ref[1] + y1_ref[...]) * dinv + b1_ref[...]
    h = jnp.where(pre > 0, pre, jnp.exp(pre) - 1.0)
    hw = jnp.dot(h, w2_ref[...], precision=lax.Precision.HIGHEST,
                 preferred_element_type=jnp.float32)
    y2_ref[...] = hw * dinv


_tc2 = pl.pallas_call(
    _tc2_body,
    grid=(N // BN,),
    in_specs=[
        pl.BlockSpec((NC, BN, D), lambda i: (0, i, 0)),
        pl.BlockSpec((BN, D), lambda i: (i, 0)),
        pl.BlockSpec((BN, WD), lambda i: (i, 0)),
        pl.BlockSpec((1, D), lambda i: (0, 0)),
        pl.BlockSpec((D, D), lambda i: (0, 0)),
    ],
    out_specs=pl.BlockSpec((BN, D), lambda i: (i, 0)),
    out_shape=jax.ShapeDtypeStruct((N, D), jnp.float32),
)


def _tc3_body(aggp_ref, y2_ref, dinv_ref, b2_ref, out_ref):
    dinv = dinv_ref[:, :1]
    out_ref[...] = (aggp_ref[0] + aggp_ref[1] + y2_ref[...]) * dinv + b2_ref[...]


_tc3 = pl.pallas_call(
    _tc3_body,
    grid=(N // BN,),
    in_specs=[
        pl.BlockSpec((NC, BN, D), lambda i: (0, i, 0)),
        pl.BlockSpec((BN, D), lambda i: (i, 0)),
        pl.BlockSpec((BN, WD), lambda i: (i, 0)),
        pl.BlockSpec((1, D), lambda i: (0, 0)),
    ],
    out_specs=pl.BlockSpec((BN, D), lambda i: (i, 0)),
    out_shape=jax.ShapeDtypeStruct((N, D), jnp.float32),
)


def kernel(x, edge_index, W1, b1, W2, b2):
    src = jnp.concatenate(
        [edge_index[0].astype(jnp.int32), jnp.zeros((EP - E,), jnp.int32)]
    ).reshape(NW * NSB, SB, K)
    dst = jnp.concatenate(
        [edge_index[1].astype(jnp.int32), jnp.full((EP - E,), NP - 1, jnp.int32)]
    ).reshape(NW * NSB, SB, K)
    zeros_nd = jnp.zeros((NP, D), jnp.float32)
    ones_kw = jnp.ones((KD, D), jnp.float32)
    b1r = b1.reshape(1, D)
    b2r = b2.reshape(1, D)

    dst_deg = edge_index[1].astype(jnp.int32).reshape(NW, CD, KD)
    degp = _deg_sc(dst_deg, zeros_nd, ones_kw)
    dinv, y1 = _tc1(x, W1, degp)
    agg1 = _agg_sc(y1, src, dst, zeros_nd)
    y2 = _tc2(agg1, y1, dinv, b1r, W2)
    agg2 = _agg_sc(y2, src, dst, zeros_nd)
    return _tc3(agg2, y2, dinv, b2r)


# speedup vs baseline: 1.0138x; 1.0138x over previous
import functools

import jax
import jax.numpy as jnp
from jax import lax
from jax.experimental import pallas as pl
from jax.experimental.pallas import tpu as pltpu
from jax.experimental.pallas import tpu_sc as plsc

N = 10000
E = 320000
D = 128
NC = 2
NS = 16
NW = NC * NS
K = 64
C = 160
EP = NW * C * K
NB = 4
SB = 40
NSB = C // SB
WD = 16
NP = 10240
RPP = NP // NS

_mesh = plsc.VectorSubcoreMesh(core_axis_name="c", subcore_axis_name="s")


KD = 80
CD = E // (NW * KD)


@functools.partial(
    pl.kernel,
    out_type=jax.ShapeDtypeStruct((NC, NP, D), jnp.float32),
    mesh=_mesh,
    scratch_types=[
        pltpu.VMEM_SHARED((NP, D), jnp.float32),
        pltpu.VMEM((CD, KD), jnp.int32),
        pltpu.VMEM((KD, D), jnp.float32),
    ],
)
def _deg_sc(dst_hbm, zeros_hbm, ones_hbm, out_hbm, acc, didx, ones_v):
    c = lax.axis_index("c")
    s = lax.axis_index("s")
    wid = s * NC + c
    pltpu.sync_copy(zeros_hbm.at[pl.ds(s * RPP, RPP)], acc.at[pl.ds(s * RPP, RPP)])
    pltpu.sync_copy(ones_hbm, ones_v)
    pltpu.sync_copy(dst_hbm.at[wid], didx)
    plsc.subcore_barrier()

    def body(j, carry):
        pltpu.sync_copy(ones_v, acc.at[didx.at[j]], add=True)
        return carry

    lax.fori_loop(0, CD, body, 0)
    plsc.subcore_barrier()
    pltpu.sync_copy(acc.at[pl.ds(s * RPP, RPP)], out_hbm.at[c, pl.ds(s * RPP, RPP)])


@functools.partial(
    pl.kernel,
    out_type=jax.ShapeDtypeStruct((NC, NP, D), jnp.float32),
    mesh=_mesh,
    scratch_types=[
        pltpu.VMEM_SHARED((NP, D), jnp.float32),
        pltpu.VMEM((SB, K), jnp.int32),
        pltpu.VMEM((SB, K), jnp.int32),
        pltpu.VMEM((NB, K, D), jnp.float32),
        [pltpu.SemaphoreType.DMA] * NB,
        [pltpu.SemaphoreType.DMA] * NB,
    ],
)
def _agg_sc(y_hbm, src_hbm, dst_hbm, zeros_hbm, out_hbm, acc, sidx, didx, rows,
            gsems, ssems):
    c = lax.axis_index("c")
    s = lax.axis_index("s")
    wid = s * NC + c
    pltpu.sync_copy(zeros_hbm.at[pl.ds(s * RPP, RPP)], acc.at[pl.ds(s * RPP, RPP)])
    plsc.subcore_barrier()

    def gwait(b):
        pltpu.make_async_copy(y_hbm.at[sidx.at[0]], rows.at[b], gsems[b]).wait()

    def swait(b):
        pltpu.make_async_copy(y_hbm.at[sidx.at[0]], rows.at[b], ssems[b]).wait()

    for t in range(NSB):
        pltpu.sync_copy(src_hbm.at[wid * NSB + t], sidx)
        pltpu.sync_copy(dst_hbm.at[wid * NSB + t], didx)
        for b in range(NB):
            pltpu.async_copy(y_hbm.at[sidx.at[b]], rows.at[b], gsems[b])

        def body(g, carry):
            for b in range(NB):
                j = g * NB + b
                gwait(b)
                pltpu.async_copy(rows.at[b], acc.at[didx.at[j]], ssems[b], add=True)
            for b in range(NB):
                j = (g + 1) * NB + b
                swait(b)
                pltpu.async_copy(y_hbm.at[sidx.at[j]], rows.at[b], gsems[b])
            return carry

        lax.fori_loop(0, SB // NB - 1, body, 0)
        for b in range(NB):
            j = SB - NB + b
            gwait(b)
            pltpu.async_copy(rows.at[b], acc.at[didx.at[j]], ssems[b], add=True)
        for b in range(NB):
            swait(b)

    plsc.subcore_barrier()
    pltpu.sync_copy(acc.at[pl.ds(s * RPP, RPP)], out_hbm.at[c, pl.ds(s * RPP, RPP)])


BN = 1000


def _tc1_body(x_ref, w_ref, degp_ref, dinv_ref, y_ref):
    deg = 1.0 + degp_ref[0, :, :1] + degp_ref[1, :, :1]
    dinv = lax.rsqrt(deg)
    dinv_ref[...] = jnp.broadcast_to(dinv, (BN, WD))
    xw = jnp.dot(x_ref[...], w_ref[...], precision=lax.Precision.HIGHEST,
                 preferred_element_type=jnp.float32)
    y_ref[...] = xw * dinv


_tc1 = pl.pallas_call(
    _tc1_body,
    grid=(N // BN,),
    in_specs=[
        pl.BlockSpec((BN, D), lambda i: (i, 0)),
        pl.BlockSpec((D, D), lambda i: (0, 0)),
        pl.BlockSpec((NC, BN, D), lambda i: (0, i, 0)),
    ],
    out_specs=[
        pl.BlockSpec((BN, WD), lambda i: (i, 0)),
        pl.BlockSpec((BN, D), lambda i: (i, 0)),
    ],
    out_shape=[
        jax.ShapeDtypeStruct((N, WD), jnp.float32),
        jax.ShapeDtypeStruct((N, D), jnp.float32),
    ],
)


def _tc2_body(aggp_ref, y1_ref, dinv_ref, b1_ref, w2_ref, y2_ref):
    dinv = dinv_ref[:, :1]
    pre = (aggp_ref[0] + aggp_ref[1] + y1_ref[...]) * dinv + b1_ref[...]
    h = jnp.where(pre > 0, pre, jnp.exp(pre) - 1.0)
    hw = jnp.dot(h, w2_ref[...], precision=lax.Precision.HIGHEST,
                 preferred_element_type=jnp.float32)
    y2_ref[...] = hw * dinv


_tc2 = pl.pallas_call(
    _tc2_body,
    grid=(N // BN,),
    in_specs=[
        pl.BlockSpec((NC, BN, D), lambda i: (0, i, 0)),
        pl.BlockSpec((BN, D), lambda i: (i, 0)),
        pl.BlockSpec((BN, WD), lambda i: (i, 0)),
        pl.BlockSpec((1, D), lambda i: (0, 0)),
        pl.BlockSpec((D, D), lambda i: (0, 0)),
    ],
    out_specs=pl.BlockSpec((BN, D), lambda i: (i, 0)),
    out_shape=jax.ShapeDtypeStruct((N, D), jnp.float32),
)


def _tc3_body(aggp_ref, y2_ref, dinv_ref, b2_ref, out_ref):
    dinv = dinv_ref[:, :1]
    out_ref[...] = (aggp_ref[0] + aggp_ref[1] + y2_ref[...]) * dinv + b2_ref[...]


_tc3 = pl.pallas_call(
    _tc3_body,
    grid=(N // BN,),
    in_specs=[
        pl.BlockSpec((NC, BN, D), lambda i: (0, i, 0)),
        pl.BlockSpec((BN, D), lambda i: (i, 0)),
        pl.BlockSpec((BN, WD), lambda i: (i, 0)),
        pl.BlockSpec((1, D), lambda i: (0, 0)),
    ],
    out_specs=pl.BlockSpec((BN, D), lambda i: (i, 0)),
    out_shape=jax.ShapeDtypeStruct((N, D), jnp.float32),
)


def kernel(x, edge_index, W1, b1, W2, b2):
    src = jnp.concatenate(
        [edge_index[0].astype(jnp.int32), jnp.zeros((EP - E,), jnp.int32)]
    ).reshape(NW * NSB, SB, K)
    pad_dst = N + (jnp.arange(EP - E, dtype=jnp.int32) % (NP - N))
    dst = jnp.concatenate(
        [edge_index[1].astype(jnp.int32), pad_dst]
    ).reshape(NW * NSB, SB, K)
    zeros_nd = jnp.zeros((NP, D), jnp.float32)
    ones_kw = jnp.ones((KD, D), jnp.float32)
    b1r = b1.reshape(1, D)
    b2r = b2.reshape(1, D)

    dst_deg = edge_index[1].astype(jnp.int32).reshape(NW, CD, KD)
    degp = _deg_sc(dst_deg, zeros_nd, ones_kw)
    dinv, y1 = _tc1(x, W1, degp)
    agg1 = _agg_sc(y1, src, dst, zeros_nd)
    y2 = _tc2(agg1, y1, dinv, b1r, W2)
    agg2 = _agg_sc(y2, src, dst, zeros_nd)
    return _tc3(agg2, y2, dinv, b2r)


# speedup vs baseline: 1.0329x; 1.0188x over previous
import functools

import jax
import jax.numpy as jnp
from jax import lax
from jax.experimental import pallas as pl
from jax.experimental.pallas import tpu as pltpu
from jax.experimental.pallas import tpu_sc as plsc

N = 10000
E = 320000
D = 128
NC = 2
NS = 16
NW = NC * NS
K = 64
C = 160
EP = NW * C * K
NB = 4
SB = 40
NSB = C // SB
WD = 16
NP = 10240
RPP = NP // NS

_mesh = plsc.VectorSubcoreMesh(core_axis_name="c", subcore_axis_name="s")


KD = 80
CD = E // (NW * KD)


@functools.partial(
    pl.kernel,
    out_type=jax.ShapeDtypeStruct((NC, NP, D), jnp.float32),
    mesh=_mesh,
    scratch_types=[
        pltpu.VMEM_SHARED((NP, D), jnp.float32),
        pltpu.VMEM((CD, KD), jnp.int32),
        pltpu.VMEM((KD, D), jnp.float32),
    ],
)
def _deg_sc(dst_hbm, zeros_hbm, ones_hbm, out_hbm, acc, didx, ones_v):
    c = lax.axis_index("c")
    s = lax.axis_index("s")
    wid = s * NC + c
    pltpu.sync_copy(zeros_hbm.at[pl.ds(s * RPP, RPP)], acc.at[pl.ds(s * RPP, RPP)])
    pltpu.sync_copy(ones_hbm, ones_v)
    pltpu.sync_copy(dst_hbm.at[wid], didx)
    plsc.subcore_barrier()

    def body(j, carry):
        pltpu.sync_copy(ones_v, acc.at[didx.at[j]], add=True)
        return carry

    lax.fori_loop(0, CD, body, 0)
    plsc.subcore_barrier()
    pltpu.sync_copy(acc.at[pl.ds(s * RPP, RPP)], out_hbm.at[c, pl.ds(s * RPP, RPP)])


@functools.partial(
    pl.kernel,
    out_type=jax.ShapeDtypeStruct((NC, NP, D), jnp.float32),
    mesh=_mesh,
    scratch_types=[
        pltpu.VMEM_SHARED((NP, D), jnp.float32),
        pltpu.VMEM((SB, K), jnp.int32),
        pltpu.VMEM((SB, K), jnp.int32),
        pltpu.VMEM((NB, K, D), jnp.float32),
        [pltpu.SemaphoreType.DMA] * NB,
        [pltpu.SemaphoreType.DMA] * NB,
    ],
)
def _agg_sc(y_hbm, src_hbm, dst_hbm, zeros_hbm, out_hbm, acc, sidx, didx, rows,
            gsems, ssems):
    c = lax.axis_index("c")
    s = lax.axis_index("s")
    wid = s * NC + c
    pltpu.sync_copy(zeros_hbm.at[pl.ds(s * RPP, RPP)], acc.at[pl.ds(s * RPP, RPP)])
    plsc.subcore_barrier()

    def gwait(b):
        pltpu.make_async_copy(y_hbm.at[sidx.at[0]], rows.at[b], gsems[b]).wait()

    def swait(b):
        pltpu.make_async_copy(y_hbm.at[sidx.at[0]], rows.at[b], ssems[b]).wait()

    for t in range(NSB):
        pltpu.sync_copy(src_hbm.at[wid * NSB + t], sidx)
        pltpu.sync_copy(dst_hbm.at[wid * NSB + t], didx)
        for b in range(NB):
            pltpu.async_copy(y_hbm.at[sidx.at[b]], rows.at[b], gsems[b])

        def body(g, carry):
            for b in range(NB):
                j = g * NB + b
                gwait(b)
                pltpu.sync_copy(rows.at[b], acc.at[didx.at[j]], add=True)
                pltpu.async_copy(y_hbm.at[sidx.at[(g + 1) * NB + b]], rows.at[b],
                                 gsems[b])
            return carry

        lax.fori_loop(0, SB // NB - 1, body, 0)
        for b in range(NB):
            j = SB - NB + b
            gwait(b)
            pltpu.sync_copy(rows.at[b], acc.at[didx.at[j]], add=True)

    plsc.subcore_barrier()
    pltpu.sync_copy(acc.at[pl.ds(s * RPP, RPP)], out_hbm.at[c, pl.ds(s * RPP, RPP)])


BN = 1000


def _tc1_body(x_ref, w_ref, degp_ref, dinv_ref, y_ref):
    deg = 1.0 + degp_ref[0, :, :1] + degp_ref[1, :, :1]
    dinv = lax.rsqrt(deg)
    dinv_ref[...] = jnp.broadcast_to(dinv, (BN, WD))
    xw = jnp.dot(x_ref[...], w_ref[...], precision=lax.Precision.HIGHEST,
                 preferred_element_type=jnp.float32)
    y_ref[...] = xw * dinv


_tc1 = pl.pallas_call(
    _tc1_body,
    grid=(N // BN,),
    in_specs=[
        pl.BlockSpec((BN, D), lambda i: (i, 0)),
        pl.BlockSpec((D, D), lambda i: (0, 0)),
        pl.BlockSpec((NC, BN, D), lambda i: (0, i, 0)),
    ],
    out_specs=[
        pl.BlockSpec((BN, WD), lambda i: (i, 0)),
        pl.BlockSpec((BN, D), lambda i: (i, 0)),
    ],
    out_shape=[
        jax.ShapeDtypeStruct((N, WD), jnp.float32),
        jax.ShapeDtypeStruct((N, D), jnp.float32),
    ],
)


def _tc2_body(aggp_ref, y1_ref, dinv_ref, b1_ref, w2_ref, y2_ref):
    dinv = dinv_ref[:, :1]
    pre = (aggp_ref[0] + aggp_ref[1] + y1_ref[...]) * dinv + b1_ref[...]
    h = jnp.where(pre > 0, pre, jnp.exp(pre) - 1.0)
    hw = jnp.dot(h, w2_ref[...], precision=lax.Precision.HIGHEST,
                 preferred_element_type=jnp.float32)
    y2_ref[...] = hw * dinv


_tc2 = pl.pallas_call(
    _tc2_body,
    grid=(N // BN,),
    in_specs=[
        pl.BlockSpec((NC, BN, D), lambda i: (0, i, 0)),
        pl.BlockSpec((BN, D), lambda i: (i, 0)),
        pl.BlockSpec((BN, WD), lambda i: (i, 0)),
        pl.BlockSpec((1, D), lambda i: (0, 0)),
        pl.BlockSpec((D, D), lambda i: (0, 0)),
    ],
    out_specs=pl.BlockSpec((BN, D), lambda i: (i, 0)),
    out_shape=jax.ShapeDtypeStruct((N, D), jnp.float32),
)


def _tc3_body(aggp_ref, y2_ref, dinv_ref, b2_ref, out_ref):
    dinv = dinv_ref[:, :1]
    out_ref[...] = (aggp_ref[0] + aggp_ref[1] + y2_ref[...]) * dinv + b2_ref[...]


_tc3 = pl.pallas_call(
    _tc3_body,
    grid=(N // BN,),
    in_specs=[
        pl.BlockSpec((NC, BN, D), lambda i: (0, i, 0)),
        pl.BlockSpec((BN, D), lambda i: (i, 0)),
        pl.BlockSpec((BN, WD), lambda i: (i, 0)),
        pl.BlockSpec((1, D), lambda i: (0, 0)),
    ],
    out_specs=pl.BlockSpec((BN, D), lambda i: (i, 0)),
    out_shape=jax.ShapeDtypeStruct((N, D), jnp.float32),
)


def kernel(x, edge_index, W1, b1, W2, b2):
    src = jnp.concatenate(
        [edge_index[0].astype(jnp.int32), jnp.zeros((EP - E,), jnp.int32)]
    ).reshape(NW * NSB, SB, K)
    pad_dst = N + (jnp.arange(EP - E, dtype=jnp.int32) % (NP - N))
    dst = jnp.concatenate(
        [edge_index[1].astype(jnp.int32), pad_dst]
    ).reshape(NW * NSB, SB, K)
    zeros_nd = jnp.zeros((NP, D), jnp.float32)
    ones_kw = jnp.ones((KD, D), jnp.float32)
    b1r = b1.reshape(1, D)
    b2r = b2.reshape(1, D)

    dst_deg = edge_index[1].astype(jnp.int32).reshape(NW, CD, KD)
    degp = _deg_sc(dst_deg, zeros_nd, ones_kw)
    dinv, y1 = _tc1(x, W1, degp)
    agg1 = _agg_sc(y1, src, dst, zeros_nd)
    y2 = _tc2(agg1, y1, dinv, b1r, W2)
    agg2 = _agg_sc(y2, src, dst, zeros_nd)
    return _tc3(agg2, y2, dinv, b2r)


# speedup vs baseline: 3.1724x; 3.0712x over previous
import functools

import jax
import jax.numpy as jnp
from jax import lax
from jax.experimental import pallas as pl
from jax.experimental.pallas import tpu as pltpu
from jax.experimental.pallas import tpu_sc as plsc

N = 10000
E = 320000
D = 128
NC = 2
NS = 16
NW = NC * NS
K = 64
C = 160
EP = NW * C * K
NB = 4
SB = 40
NSB = C // SB
WD = 16
NP = 10240
RPP = NP // NS

_mesh = plsc.VectorSubcoreMesh(core_axis_name="c", subcore_axis_name="s")


KD = 80
CD = E // (NW * KD)


@functools.partial(
    pl.kernel,
    out_type=jax.ShapeDtypeStruct((NC, NP, D), jnp.float32),
    mesh=_mesh,
    scratch_types=[
        pltpu.VMEM_SHARED((NP, D), jnp.float32),
        pltpu.VMEM((CD, KD), jnp.int32),
        pltpu.VMEM((KD, D), jnp.float32),
    ],
)
def _deg_sc(dst_hbm, zeros_hbm, ones_hbm, out_hbm, acc, didx, ones_v):
    c = lax.axis_index("c")
    s = lax.axis_index("s")
    wid = s * NC + c
    pltpu.sync_copy(zeros_hbm.at[pl.ds(s * RPP, RPP)], acc.at[pl.ds(s * RPP, RPP)])
    pltpu.sync_copy(ones_hbm, ones_v)
    pltpu.sync_copy(dst_hbm.at[wid], didx)
    plsc.subcore_barrier()

    def body(j, carry):
        pltpu.sync_copy(ones_v, acc.at[didx.at[j]], add=True)
        return carry

    lax.fori_loop(0, CD, body, 0)
    plsc.subcore_barrier()
    pltpu.sync_copy(acc.at[pl.ds(s * RPP, RPP)], out_hbm.at[c, pl.ds(s * RPP, RPP)])


@functools.partial(
    pl.kernel,
    out_type=jax.ShapeDtypeStruct((NC, NP, D), jnp.float32),
    mesh=_mesh,
    scratch_types=[
        pltpu.VMEM_SHARED((NP, D), jnp.float32),
        pltpu.VMEM((SB, K), jnp.int32),
        pltpu.VMEM((SB, K), jnp.int32),
        pltpu.VMEM((NB, K, D), jnp.float32),
        [pltpu.SemaphoreType.DMA] * NB,
        [pltpu.SemaphoreType.DMA] * NB,
    ],
)
def _agg_sc(y_hbm, src_hbm, dst_hbm, zeros_hbm, out_hbm, acc, sidx, didx, rows,
            gsems, ssems):
    c = lax.axis_index("c")
    s = lax.axis_index("s")
    wid = s * NC + c
    pltpu.sync_copy(zeros_hbm.at[pl.ds(s * RPP, RPP)], acc.at[pl.ds(s * RPP, RPP)])
    plsc.subcore_barrier()

    def gwait(b):
        pltpu.make_async_copy(y_hbm.at[sidx.at[0]], rows.at[b], gsems[b]).wait()

    def swait(b):
        pltpu.make_async_copy(y_hbm.at[sidx.at[0]], rows.at[b], ssems[b]).wait()

    for t in range(NSB):
        pltpu.sync_copy(src_hbm.at[wid * NSB + t], sidx)
        pltpu.sync_copy(dst_hbm.at[wid * NSB + t], didx)
        for b in range(NB):
            pltpu.async_copy(y_hbm.at[sidx.at[b]], rows.at[b], gsems[b])

        def body(g, carry):
            for b in range(NB):
                j = g * NB + b
                gwait(b)
                pltpu.sync_copy(rows.at[b], acc.at[didx.at[j]], add=True)
                pltpu.async_copy(y_hbm.at[sidx.at[(g + 1) * NB + b]], rows.at[b],
                                 gsems[b])
            return carry

        lax.fori_loop(0, SB // NB - 1, body, 0)
        for b in range(NB):
            j = SB - NB + b
            gwait(b)
            pltpu.sync_copy(rows.at[b], acc.at[didx.at[j]], add=True)

    plsc.subcore_barrier()
    pltpu.sync_copy(acc.at[pl.ds(s * RPP, RPP)], out_hbm.at[c, pl.ds(s * RPP, RPP)])


BN = 1000


def _tc1_body(x_ref, w_ref, degp_ref, dinv_ref, y_ref):
    deg = 1.0 + degp_ref[0, :, :1] + degp_ref[1, :, :1]
    dinv = lax.rsqrt(deg)
    dinv_ref[...] = jnp.broadcast_to(dinv, (BN, WD))
    xw = jnp.dot(x_ref[...], w_ref[...], precision=lax.Precision.HIGHEST,
                 preferred_element_type=jnp.float32)
    y_ref[...] = xw * dinv


_tc1 = pl.pallas_call(
    _tc1_body,
    grid=(N // BN,),
    in_specs=[
        pl.BlockSpec((BN, D), lambda i: (i, 0)),
        pl.BlockSpec((D, D), lambda i: (0, 0)),
        pl.BlockSpec((NC, BN, D), lambda i: (0, i, 0)),
    ],
    out_specs=[
        pl.BlockSpec((BN, WD), lambda i: (i, 0)),
        pl.BlockSpec((BN, D), lambda i: (i, 0)),
    ],
    out_shape=[
        jax.ShapeDtypeStruct((N, WD), jnp.float32),
        jax.ShapeDtypeStruct((N, D), jnp.float32),
    ],
)


def _tc2_body(aggp_ref, y1_ref, dinv_ref, b1_ref, w2_ref, y2_ref):
    dinv = dinv_ref[:, :1]
    pre = (aggp_ref[0] + aggp_ref[1] + y1_ref[...]) * dinv + b1_ref[...]
    h = jnp.where(pre > 0, pre, jnp.exp(pre) - 1.0)
    hw = jnp.dot(h, w2_ref[...], precision=lax.Precision.HIGHEST,
                 preferred_element_type=jnp.float32)
    y2_ref[...] = hw * dinv


_tc2 = pl.pallas_call(
    _tc2_body,
    grid=(N // BN,),
    in_specs=[
        pl.BlockSpec((NC, BN, D), lambda i: (0, i, 0)),
        pl.BlockSpec((BN, D), lambda i: (i, 0)),
        pl.BlockSpec((BN, WD), lambda i: (i, 0)),
        pl.BlockSpec((1, D), lambda i: (0, 0)),
        pl.BlockSpec((D, D), lambda i: (0, 0)),
    ],
    out_specs=pl.BlockSpec((BN, D), lambda i: (i, 0)),
    out_shape=jax.ShapeDtypeStruct((N, D), jnp.float32),
)


def _tc3_body(aggp_ref, y2_ref, dinv_ref, b2_ref, out_ref):
    dinv = dinv_ref[:, :1]
    out_ref[...] = (aggp_ref[0] + aggp_ref[1] + y2_ref[...]) * dinv + b2_ref[...]


_tc3 = pl.pallas_call(
    _tc3_body,
    grid=(N // BN,),
    in_specs=[
        pl.BlockSpec((NC, BN, D), lambda i: (0, i, 0)),
        pl.BlockSpec((BN, D), lambda i: (i, 0)),
        pl.BlockSpec((BN, WD), lambda i: (i, 0)),
        pl.BlockSpec((1, D), lambda i: (0, 0)),
    ],
    out_specs=pl.BlockSpec((BN, D), lambda i: (i, 0)),
    out_shape=jax.ShapeDtypeStruct((N, D), jnp.float32),
)


def kernel(x, edge_index, W1, b1, W2, b2):
    pad_src = jnp.arange(EP - E, dtype=jnp.int32) % N
    src = jnp.concatenate(
        [edge_index[0].astype(jnp.int32), pad_src]
    ).reshape(NW * NSB, SB, K)
    pad_dst = N + (jnp.arange(EP - E, dtype=jnp.int32) % (NP - N))
    dst = jnp.concatenate(
        [edge_index[1].astype(jnp.int32), pad_dst]
    ).reshape(NW * NSB, SB, K)
    zeros_nd = jnp.zeros((NP, D), jnp.float32)
    ones_kw = jnp.ones((KD, D), jnp.float32)
    b1r = b1.reshape(1, D)
    b2r = b2.reshape(1, D)

    dst_deg = edge_index[1].astype(jnp.int32).reshape(NW, CD, KD)
    degp = _deg_sc(dst_deg, zeros_nd, ones_kw)
    dinv, y1 = _tc1(x, W1, degp)
    agg1 = _agg_sc(y1, src, dst, zeros_nd)
    y2 = _tc2(agg1, y1, dinv, b1r, W2)
    agg2 = _agg_sc(y2, src, dst, zeros_nd)
    return _tc3(agg2, y2, dinv, b2r)
